# Initial kernel scaffold; baseline (speedup 1.0000x reference)
import jax
import jax.numpy as jnp
from jax.experimental import pallas as pl


def kernel(x, W):
    raise NotImplementedError("write your pallas kernel here")


# speedup vs baseline: 2.7509x; 2.7509x over previous
import functools

import jax
import jax.numpy as jnp
from jax import lax
from jax.experimental import pallas as pl
from jax.experimental.pallas import tpu as pltpu
from jax.experimental.pallas import tpu_sc as plsc

HASH_LENGTH = 32

_L = 16
_NW = 32
_CHUNK = 128
_CAP = 4096

_BB_A = 128
_OB = 512


def _mm_kernel(x_ref, w_ref, act_ref):
    act_ref[...] = lax.dot_general(
        x_ref[...], w_ref[...], (((1,), (1,)), ((), ())),
        preferred_element_type=jnp.float32)


def _cm_kernel(act_ref, cm_ref):
    a = act_ref[...]
    r = a.reshape(a.shape[0], a.shape[1] // _CHUNK, _CHUNK)
    cm_ref[...] = jnp.max(r, axis=2)


def _bcast_last(v):
    return v.at[jnp.full((_L,), _L - 1, jnp.int32)].get(
        mode="promise_in_bounds")


def _rank_k_bits(count_fn, k):
    def probe_body(s, t):
        pb = t | lax.shift_left(jnp.int32(1), jnp.int32(30) - s)
        return jnp.where(count_fn(pb) >= k, pb, t)
    return lax.fori_loop(0, 31, probe_body, jnp.int32(0))


def _make_sc_topk(B, O, interpret=False):
    rows_pw = B // _NW
    nch = O // _CHUNK
    nvec = O // _L
    mesh = plsc.VectorSubcoreMesh(core_axis_name="c", subcore_axis_name="s",
                                  num_cores=2, num_subcores=16)

    def body(act_hbm, cm_hbm, out_hbm,
             row_buf, out_stage, cm_all, cand_v, cand_i,
             kept_v, kept_i, prev_i, in_sem, out_sem):
        wid = lax.axis_index("s") * 2 + lax.axis_index("c")
        r0 = wid * rows_pw
        iota = lax.iota(jnp.int32, _L)
        ones = jnp.ones((_L,), jnp.int32)
        zeros_i = jnp.zeros((_L,), jnp.int32)


        def zs(j, c):
            out_stage[0, pl.ds(j * _L, _L)] = zeros_i
            out_stage[1, pl.ds(j * _L, _L)] = zeros_i
            return c
        lax.fori_loop(0, nvec, zs, 0)
        for p in range(2):
            for q in range(2):
                prev_i[p, pl.ds(q * _L, _L)] = zeros_i

        pltpu.sync_copy(cm_hbm.at[pl.ds(r0, rows_pw)], cm_all)
        pltpu.async_copy(act_hbm.at[r0], row_buf.at[0], in_sem.at[0])
        pltpu.async_copy(act_hbm.at[r0 + 1], row_buf.at[1], in_sem.at[1])

        def row_body(i, c):
            p = lax.rem(i, 2)
            r = r0 + i
            pltpu.make_async_copy(
                act_hbm.at[r], row_buf.at[p], in_sem.at[p]).wait()

            def cm_count(pb):
                pbv = jnp.full((_L,), pb, jnp.int32)
                def cnt(j, acc):
                    v = cm_all[i, pl.ds(j * _L, _L)]
                    return acc + jnp.where(v >= pbv, ones, zeros_i)
                return jnp.sum(lax.fori_loop(0, nch // _L, cnt, zeros_i))
            tcm = _rank_k_bits(cm_count, HASH_LENGTH)

            tcm_v = jnp.full((_L,), tcm, jnp.int32)
            def scan_body(j, off):
                v = row_buf[p, pl.ds(j * _L, _L)]
                m = v >= tcm_v
                csum = plsc.cumsum(jnp.where(m, ones, zeros_i))
                pos = jnp.minimum(off + csum - 1, _CAP - 1)
                plsc.store_scatter(cand_v, [pos], v, mask=m)
                plsc.store_scatter(cand_i, [pos], iota + j * _L, mask=m)
                return off + _bcast_last(csum)
            offv = lax.fori_loop(0, nvec, scan_body, zeros_i)
            off = jnp.minimum(jnp.max(offv), jnp.int32(_CAP))
            cand_v[pl.ds(off, _L)] = jnp.full((_L,), -1, jnp.int32)
            nv = (off + _L - 1) // _L

            def cand_count(pb):
                pbv = jnp.full((_L,), pb, jnp.int32)
                def cnt(j, acc):
                    v = cand_v[pl.ds(j * _L, _L)]
                    return acc + jnp.where(v >= pbv, ones, zeros_i)
                return jnp.sum(lax.fori_loop(0, nv, cnt, zeros_i))
            tb = _rank_k_bits(cand_count, HASH_LENGTH)
            tbv = jnp.full((_L,), tb, jnp.int32)

            def cnt_strict(j, acc):
                v = cand_v[pl.ds(j * _L, _L)]
                return acc + jnp.where(v > tbv, ones, zeros_i)
            nstrict = jnp.sum(lax.fori_loop(0, nv, cnt_strict, zeros_i))
            quota_v = jnp.full((_L,), HASH_LENGTH - nstrict, jnp.int32)

            def sel_body(j, carry):
                koff, eqtot = carry
                v = cand_v[pl.ds(j * _L, _L)]
                ci = cand_i[pl.ds(j * _L, _L)]
                gt = v > tbv
                eq = v == tbv
                eqc = plsc.cumsum(jnp.where(eq, ones, zeros_i))
                keep = gt | (eq & ((eqc + eqtot) <= quota_v))
                kc = plsc.cumsum(jnp.where(keep, ones, zeros_i))
                pos = jnp.minimum(koff + kc - 1,
                                  jnp.int32(HASH_LENGTH + _L - 1))
                plsc.store_scatter(kept_v, [pos], v, mask=keep)
                plsc.store_scatter(kept_i, [pos], ci, mask=keep)
                return koff + _bcast_last(kc), eqtot + _bcast_last(eqc)
            lax.fori_loop(0, nv, sel_body, (zeros_i, zeros_i))

            @pl.when(i >= 2)
            def _():
                pltpu.make_async_copy(
                    out_stage.at[p], out_hbm.at[r - 2], out_sem.at[p]).wait()
            pfull = jnp.full((_L,), p, jnp.int32)
            for q in range(2):
                ki = prev_i[p, pl.ds(q * _L, _L)]
                plsc.store_scatter(out_stage, [pfull, ki], zeros_i)
            for q in range(2):
                ki = kept_i[pl.ds(q * _L, _L)]
                kv = kept_v[pl.ds(q * _L, _L)]
                plsc.store_scatter(out_stage, [pfull, ki], kv)
                prev_i[p, pl.ds(q * _L, _L)] = ki
            pltpu.async_copy(out_stage.at[p], out_hbm.at[r], out_sem.at[p])

            @pl.when(i + 2 < rows_pw)
            def _():
                pltpu.async_copy(
                    act_hbm.at[r + 2], row_buf.at[p], in_sem.at[p])
            return c
        lax.fori_loop(0, rows_pw, row_body, 0)

        pltpu.make_async_copy(
            out_stage.at[0], out_hbm.at[r0 + rows_pw - 2], out_sem.at[0]).wait()
        pltpu.make_async_copy(
            out_stage.at[1], out_hbm.at[r0 + rows_pw - 1], out_sem.at[1]).wait()

    return pl.kernel(
        body,
        out_type=jax.ShapeDtypeStruct((B, O), jnp.int32),
        mesh=mesh,
        interpret=interpret,
        compiler_params=pltpu.CompilerParams(needs_layout_passes=False),
        scratch_types=[
            pltpu.VMEM((2, O), jnp.int32),
            pltpu.VMEM((2, O), jnp.int32),
            pltpu.VMEM((rows_pw, nch), jnp.int32),
            pltpu.VMEM((_CAP + _L,), jnp.int32),
            pltpu.VMEM((_CAP + _L,), jnp.int32),
            pltpu.VMEM((HASH_LENGTH + _L,), jnp.int32),
            pltpu.VMEM((HASH_LENGTH + _L,), jnp.int32),
            pltpu.VMEM((2, 2 * _L), jnp.int32),
            pltpu.SemaphoreType.DMA((2,)),
            pltpu.SemaphoreType.DMA((2,)),
        ],
    )


def _matmul_chunkmax(x, W):
    B, K = x.shape
    O = W.shape[0]
    act = pl.pallas_call(
        _mm_kernel,
        grid=(O // _OB, B // _BB_A),
        in_specs=[
            pl.BlockSpec((_BB_A, K), lambda o, b: (b, 0)),
            pl.BlockSpec((_OB, K), lambda o, b: (o, 0)),
        ],
        out_specs=pl.BlockSpec((_BB_A, _OB), lambda o, b: (b, o)),
        out_shape=jax.ShapeDtypeStruct((B, O), jnp.float32),
    )(x, W)
    cm = pl.pallas_call(
        _cm_kernel,
        grid=(B // _BB_A,),
        in_specs=[pl.BlockSpec((_BB_A, O), lambda b: (b, 0))],
        out_specs=pl.BlockSpec((_BB_A, O // _CHUNK), lambda b: (b, 0)),
        out_shape=jax.ShapeDtypeStruct((B, O // _CHUNK), jnp.float32),
    )(act)
    return act, cm


def kernel(x, W):
    B, _ = x.shape
    O = W.shape[0]
    act, cm = _matmul_chunkmax(x, W)
    act_i = lax.bitcast_convert_type(act, jnp.int32)
    cm_i = lax.bitcast_convert_type(cm, jnp.int32)
    out_i = _make_sc_topk(B, O)(act_i, cm_i)
    return lax.bitcast_convert_type(out_i, jnp.float32)


# speedup vs baseline: 6.3700x; 2.3156x over previous
import functools

import jax
import jax.numpy as jnp
from jax import lax
from jax.experimental import pallas as pl
from jax.experimental.pallas import tpu as pltpu
from jax.experimental.pallas import tpu_sc as plsc

HASH_LENGTH = 32

_L = 16
_NW = 32
_CHUNK = 128
_CAP = 4096
_NSTAT = 8

_BB_A = 256
_OB = 2560


def _mm_kernel(x_ref, w_ref, act_ref):
    act_ref[...] = lax.dot_general(
        x_ref[...].astype(jnp.bfloat16), w_ref[...].astype(jnp.bfloat16),
        (((1,), (1,)), ((), ())),
        preferred_element_type=jnp.float32)


def _cm_kernel(act_ref, cm_ref):
    a = act_ref[...]
    r = a.reshape(a.shape[0], a.shape[1] // _CHUNK, _CHUNK)
    cm_ref[...] = jnp.max(r, axis=2)


def _bcast_last(v):
    return v.at[jnp.full((_L,), _L - 1, jnp.int32)].get(
        mode="promise_in_bounds")


def _rank_k_bits(count_fn, k):
    def probe_body(s, t):
        pb = t | lax.shift_left(jnp.int32(1), jnp.int32(30) - s)
        return jnp.where(count_fn(pb) >= k, pb, t)
    return lax.fori_loop(0, 31, probe_body, jnp.int32(0))


def _make_sc_topk(B, O, interpret=False):
    rows_pw = B // _NW
    nch = O // _CHUNK
    nvec = O // _L
    mesh = plsc.VectorSubcoreMesh(core_axis_name="c", subcore_axis_name="s",
                                  num_cores=2, num_subcores=16)

    def body(act_hbm, cm_hbm, out_hbm,
             row_buf, out_stage, cm_all, cand_v, cand_i,
             kept_v, kept_i, prev_i, in_sem, out_sem):
        wid = lax.axis_index("s") * 2 + lax.axis_index("c")
        r0 = wid * rows_pw
        iota = lax.iota(jnp.int32, _L)
        ones = jnp.ones((_L,), jnp.int32)
        zeros_i = jnp.zeros((_L,), jnp.int32)


        def zs(j, c):
            out_stage[0, pl.ds(j * _L, _L)] = zeros_i
            out_stage[1, pl.ds(j * _L, _L)] = zeros_i
            return c
        lax.fori_loop(0, nvec, zs, 0)
        for p in range(2):
            for q in range(2):
                prev_i[p, pl.ds(q * _L, _L)] = zeros_i

        pltpu.sync_copy(cm_hbm.at[pl.ds(r0, rows_pw)], cm_all)
        pltpu.async_copy(act_hbm.at[r0], row_buf.at[0], in_sem.at[0])
        pltpu.async_copy(act_hbm.at[r0 + 1], row_buf.at[1], in_sem.at[1])

        def row_body(i, c):
            p = lax.rem(i, 2)
            r = r0 + i
            pltpu.make_async_copy(
                act_hbm.at[r], row_buf.at[p], in_sem.at[p]).wait()

            def cm_count(pb):
                pbv = jnp.full((_L,), pb, jnp.int32)
                acc = zeros_i
                for j in range(nch // _L):
                    v = cm_all[i, pl.ds(j * _L, _L)]
                    acc = acc + jnp.where(v >= pbv, ones, zeros_i)
                return jnp.sum(acc)
            def cm_probe(s, t):
                pb = t | lax.shift_left(jnp.int32(1), jnp.int32(30) - s)
                return jnp.where(cm_count(pb) >= HASH_LENGTH, pb, t)
            tcm = lax.fori_loop(0, 16, cm_probe, jnp.int32(0))

            for q in range(_NSTAT + 1):
                cand_v[pl.ds(q * _L, _L)] = jnp.full((_L,), -1, jnp.int32)

            tcm_v = jnp.full((_L,), tcm, jnp.int32)
            def scan_group(g, off):
                base = g * (8 * _L)
                vs = [row_buf[p, pl.ds(base + q * _L, _L)] for q in range(8)]
                ms = [v >= tcm_v for v in vs]
                anym = (ms[0] | ms[1]) | (ms[2] | ms[3])
                anym = anym | ((ms[4] | ms[5]) | (ms[6] | ms[7]))
                def hit(off):
                    for q in range(8):
                        m = ms[q]
                        csum = plsc.cumsum(jnp.where(m, ones, zeros_i))
                        pos = jnp.minimum(off + csum - 1, _CAP - 1)
                        plsc.store_scatter(cand_v, [pos], vs[q], mask=m)
                        plsc.store_scatter(cand_i, [pos],
                                           iota + (base + q * _L), mask=m)
                        off = off + _bcast_last(csum)
                    return off
                return lax.cond(jnp.any(anym), hit, lambda o: o, off)
            offv = lax.fori_loop(0, nvec // 8, scan_group, zeros_i)
            off = jnp.minimum(jnp.max(offv), jnp.int32(_CAP))
            cand_v[pl.ds(off, _L)] = jnp.full((_L,), -1, jnp.int32)
            nv = (off + _L - 1) // _L

            def cand_count(pb):
                pbv = jnp.full((_L,), pb, jnp.int32)
                acc = zeros_i
                for q in range(_NSTAT):
                    v = cand_v[pl.ds(q * _L, _L)]
                    acc = acc + jnp.where(v >= pbv, ones, zeros_i)
                def cnt(j, a):
                    v = cand_v[pl.ds(j * _L, _L)]
                    return a + jnp.where(v >= pbv, ones, zeros_i)
                return jnp.sum(lax.fori_loop(_NSTAT, nv, cnt, acc))
            tb = _rank_k_bits(cand_count, HASH_LENGTH)
            tbv = jnp.full((_L,), tb, jnp.int32)

            acc = zeros_i
            for q in range(_NSTAT):
                v = cand_v[pl.ds(q * _L, _L)]
                acc = acc + jnp.where(v > tbv, ones, zeros_i)
            def cnt_strict(j, a):
                v = cand_v[pl.ds(j * _L, _L)]
                return a + jnp.where(v > tbv, ones, zeros_i)
            nstrict = jnp.sum(lax.fori_loop(_NSTAT, nv, cnt_strict, acc))
            quota_v = jnp.full((_L,), HASH_LENGTH - nstrict, jnp.int32)

            def sel_one(j_static, j_dyn, carry):
                koff, eqtot = carry
                sl = (pl.ds(j_static * _L, _L) if j_dyn is None
                      else pl.ds(j_dyn * _L, _L))
                v = cand_v[sl]
                ci = cand_i[sl]
                gt = v > tbv
                eq = v == tbv
                eqc = plsc.cumsum(jnp.where(eq, ones, zeros_i))
                keep = gt | (eq & ((eqc + eqtot) <= quota_v))
                kc = plsc.cumsum(jnp.where(keep, ones, zeros_i))
                pos = jnp.minimum(koff + kc - 1,
                                  jnp.int32(HASH_LENGTH + _L - 1))
                plsc.store_scatter(kept_v, [pos], v, mask=keep)
                plsc.store_scatter(kept_i, [pos], ci, mask=keep)
                return koff + _bcast_last(kc), eqtot + _bcast_last(eqc)
            carry = (zeros_i, zeros_i)
            for q in range(_NSTAT):
                carry = sel_one(q, None, carry)
            carry = lax.fori_loop(
                _NSTAT, nv, lambda j, c: sel_one(0, j, c), carry)

            @pl.when(i >= 2)
            def _():
                pltpu.make_async_copy(
                    out_stage.at[p], out_hbm.at[r - 2], out_sem.at[p]).wait()
            pfull = jnp.full((_L,), p, jnp.int32)
            for q in range(2):
                ki = prev_i[p, pl.ds(q * _L, _L)]
                plsc.store_scatter(out_stage, [pfull, ki], zeros_i)
            for q in range(2):
                ki = kept_i[pl.ds(q * _L, _L)]
                kv = kept_v[pl.ds(q * _L, _L)]
                plsc.store_scatter(out_stage, [pfull, ki], kv)
                prev_i[p, pl.ds(q * _L, _L)] = ki
            pltpu.async_copy(out_stage.at[p], out_hbm.at[r], out_sem.at[p])

            @pl.when(i + 2 < rows_pw)
            def _():
                pltpu.async_copy(
                    act_hbm.at[r + 2], row_buf.at[p], in_sem.at[p])
            return c
        lax.fori_loop(0, rows_pw, row_body, 0)

        pltpu.make_async_copy(
            out_stage.at[0], out_hbm.at[r0 + rows_pw - 2], out_sem.at[0]).wait()
        pltpu.make_async_copy(
            out_stage.at[1], out_hbm.at[r0 + rows_pw - 1], out_sem.at[1]).wait()

    return pl.kernel(
        body,
        out_type=jax.ShapeDtypeStruct((B, O), jnp.int32),
        mesh=mesh,
        interpret=interpret,
        compiler_params=pltpu.CompilerParams(needs_layout_passes=False),
        scratch_types=[
            pltpu.VMEM((2, O), jnp.int32),
            pltpu.VMEM((2, O), jnp.int32),
            pltpu.VMEM((rows_pw, nch), jnp.int32),
            pltpu.VMEM((_CAP + _L,), jnp.int32),
            pltpu.VMEM((_CAP + _L,), jnp.int32),
            pltpu.VMEM((HASH_LENGTH + _L,), jnp.int32),
            pltpu.VMEM((HASH_LENGTH + _L,), jnp.int32),
            pltpu.VMEM((2, 2 * _L), jnp.int32),
            pltpu.SemaphoreType.DMA((2,)),
            pltpu.SemaphoreType.DMA((2,)),
        ],
    )


def _matmul_chunkmax(x, W):
    B, K = x.shape
    O = W.shape[0]
    act = pl.pallas_call(
        _mm_kernel,
        grid=(O // _OB, B // _BB_A),
        in_specs=[
            pl.BlockSpec((_BB_A, K), lambda o, b: (b, 0)),
            pl.BlockSpec((_OB, K), lambda o, b: (o, 0)),
        ],
        out_specs=pl.BlockSpec((_BB_A, _OB), lambda o, b: (b, o)),
        out_shape=jax.ShapeDtypeStruct((B, O), jnp.float32),
    )(x, W)
    bb_c = 128
    cm = pl.pallas_call(
        _cm_kernel,
        grid=(B // bb_c,),
        in_specs=[pl.BlockSpec((bb_c, O), lambda b: (b, 0))],
        out_specs=pl.BlockSpec((bb_c, O // _CHUNK), lambda b: (b, 0)),
        out_shape=jax.ShapeDtypeStruct((B, O // _CHUNK), jnp.float32),
    )(act)
    return act, cm


def kernel(x, W):
    B, _ = x.shape
    O = W.shape[0]
    act, cm = _matmul_chunkmax(x, W)
    act_i = lax.bitcast_convert_type(act, jnp.int32)
    cm_i = lax.bitcast_convert_type(cm, jnp.int32)
    out_i = _make_sc_topk(B, O)(act_i, cm_i)
    return lax.bitcast_convert_type(out_i, jnp.float32)
